# baseline (speedup 1.0000x reference)
import functools

import jax
import jax.numpy as jnp
from jax import lax
from jax.experimental import pallas as pl
from jax.experimental.pallas import tpu as pltpu
from jax.experimental.pallas import tpu_sc as plsc

E = 8
K = 2
D = 1024
DP = D // 2
F = 2048
T = 2048
C = 640
EC = E * C
BUF_ROWS = EC + 128
DUMP = EC

TB = 512
NBLK = T // TB

NW = 32
TPW = T // NW

CBLK = 640


def _b16(x):
    xi = lax.bitcast_convert_type(x, jnp.int32)
    r = xi + jnp.int32(0x7FFF) + ((xi >> 16) & 1)
    return (r >> 16) & jnp.int32(0xFFFF)


def _pack(v):
    return (_b16(v[:, :DP]) << 16) | _b16(v[:, DP:])


def _unpack(p):
    hi = lax.bitcast_convert_type(p & jnp.int32(-65536), jnp.float32)
    lo = lax.bitcast_convert_type(p << 16, jnp.float32)
    return jnp.concatenate([hi, lo], axis=1)


def _router_body(x_ref, wg_ref, d0_ref, d1_ref, c0_ref, c1_ref, w0_ref, w1_ref,
                 xp_ref, carry_ref):
    b = pl.program_id(0)

    @pl.when(b == 0)
    def _init():
        carry_ref[...] = jnp.zeros_like(carry_ref)

    xb = x_ref[...]
    xp_ref[...] = _pack(xb)

    wg = wg_ref[...]
    logits = jnp.dot(xb, wg, preferred_element_type=jnp.float32)
    lt = logits.T
    rows = lax.broadcasted_iota(jnp.int32, (128, TB), 0)
    valid = rows < E

    lm = jnp.where(valid, lt, jnp.float32(-1e30))
    m = jnp.max(lm, axis=0, keepdims=True)
    p = jnp.where(valid, jnp.exp(lt - m), 0.0)
    p = p / jnp.sum(p, axis=0, keepdims=True)

    v1 = jnp.max(jnp.where(valid, p, -1.0), axis=0, keepdims=True)
    i1 = jnp.min(jnp.where(valid & (p == v1), rows, 128), axis=0, keepdims=True)
    m2 = valid & (rows != i1)
    v2 = jnp.max(jnp.where(m2, p, -1.0), axis=0, keepdims=True)
    i2 = jnp.min(jnp.where(m2 & (p == v2), rows, 128), axis=0, keepdims=True)

    ws = v1 + v2 + 1e-9
    w0 = v1 / ws
    w1 = v2 / ws

    oh0 = (rows == i1).astype(jnp.float32)
    oh1 = (rows == i2).astype(jnp.float32)
    oh = oh0 + oh1

    r2 = lax.broadcasted_iota(jnp.int32, (TB, TB), 0)
    c2 = lax.broadcasted_iota(jnp.int32, (TB, TB), 1)
    su = (r2 < c2).astype(jnp.float32)
    base = jnp.dot(oh, su, preferred_element_type=jnp.float32)
    base = base + carry_ref[...]
    carry_ref[...] = carry_ref[...] + jnp.sum(oh, axis=1, keepdims=True)

    pos0 = jnp.sum(oh0 * base, axis=0, keepdims=True).astype(jnp.int32)
    pos1 = jnp.sum(oh1 * base, axis=0, keepdims=True).astype(jnp.int32)
    keep0 = pos0 < C
    keep1 = pos1 < C

    d0 = jnp.where(keep0, i1 * C + pos0, DUMP)
    d1 = jnp.where(keep1, i2 * C + pos1, DUMP)
    c0 = jnp.where(keep0, d0, 0)
    c1 = jnp.where(keep1, d1, 0)
    w0k = jnp.where(keep0, w0, 0.0)
    w1k = jnp.where(keep1, w1, 0.0)

    d0_ref[...] = d0.reshape(1, 1, TB)
    d1_ref[...] = d1.reshape(1, 1, TB)
    c0_ref[...] = c0.reshape(1, 1, TB)
    c1_ref[...] = c1.reshape(1, 1, TB)
    w0_ref[...] = w0k.reshape(1, 1, TB)
    w1_ref[...] = w1k.reshape(1, 1, TB)


def _router(x, wgp):
    out3i = jax.ShapeDtypeStruct((NBLK, 1, TB), jnp.int32)
    out3f = jax.ShapeDtypeStruct((NBLK, 1, TB), jnp.float32)
    spec3 = pl.BlockSpec((1, 1, TB), lambda i: (i, 0, 0))
    return pl.pallas_call(
        _router_body,
        grid=(NBLK,),
        in_specs=[
            pl.BlockSpec((TB, D), lambda i: (i, 0)),
            pl.BlockSpec((D, 128), lambda i: (0, 0)),
        ],
        out_specs=[spec3, spec3, spec3, spec3, spec3, spec3,
                   pl.BlockSpec((TB, DP), lambda i: (i, 0))],
        out_shape=[out3i, out3i, out3i, out3i, out3f, out3f,
                   jax.ShapeDtypeStruct((T, DP), jnp.int32)],
        scratch_shapes=[pltpu.VMEM((128, 1), jnp.float32)],
    )(x, wgp)


def _disp_body(xp_hbm, d0_hbm, d1_hbm, buf_hbm, d0_v, d1_v, rows_v, sem):
    wid = lax.axis_index("s") * 2 + lax.axis_index("c")
    t0 = wid * TPW
    b = t0 // TB
    off = t0 % TB
    pltpu.sync_copy(d0_hbm.at[b, 0, pl.ds(off, TPW)], d0_v)
    pltpu.sync_copy(d1_hbm.at[b, 0, pl.ds(off, TPW)], d1_v)
    pltpu.sync_copy(xp_hbm.at[pl.ds(t0, TPW)], rows_v)
    a0 = pltpu.async_copy(rows_v, buf_hbm.at[d0_v], sem)
    a1 = pltpu.async_copy(rows_v, buf_hbm.at[d1_v], sem)
    a0.wait()
    a1.wait()


def _dispatch(xp, d0, d1):
    mesh = plsc.VectorSubcoreMesh(core_axis_name="c", subcore_axis_name="s")
    return pl.kernel(
        _disp_body,
        out_type=jax.ShapeDtypeStruct((BUF_ROWS, DP), jnp.int32),
        mesh=mesh,
        scratch_types=[
            pltpu.VMEM((TPW,), jnp.int32),
            pltpu.VMEM((TPW,), jnp.int32),
            pltpu.VMEM((TPW, DP), jnp.int32),
            pltpu.SemaphoreType.DMA,
        ],
    )(xp, d0, d1)


FSP = 2
FCH = F // FSP


def _ffn_body(buf_ref, w1_ref, b1_ref, w2_ref, b2_ref, out_ref, acc_ref):
    fb = pl.program_id(1)
    xb = _unpack(buf_ref[...])
    h = jnp.dot(xb, w1_ref[0], preferred_element_type=jnp.float32) + b1_ref[0]
    h = jnp.maximum(h, 0.0)
    part = jnp.dot(h, w2_ref[0], preferred_element_type=jnp.float32)

    @pl.when(fb == 0)
    def _first():
        acc_ref[...] = part + b2_ref[0]

    @pl.when(fb > 0)
    def _rest():
        acc_ref[...] = acc_ref[...] + part

    @pl.when(fb == FSP - 1)
    def _emit():
        out_ref[...] = _pack(acc_ref[...])


def _ffn(buf, W1, b1r, W2, b2r):
    return pl.pallas_call(
        _ffn_body,
        grid=(E, FSP),
        in_specs=[
            pl.BlockSpec((CBLK, DP), lambda e, fb: (e, 0)),
            pl.BlockSpec((1, D, FCH), lambda e, fb: (e, 0, fb)),
            pl.BlockSpec((1, 1, FCH), lambda e, fb: (e, 0, fb)),
            pl.BlockSpec((1, FCH, D), lambda e, fb: (e, fb, 0)),
            pl.BlockSpec((1, 1, D), lambda e, fb: (e, 0, 0)),
        ],
        out_specs=pl.BlockSpec((CBLK, DP), lambda e, fb: (e, 0)),
        out_shape=jax.ShapeDtypeStruct((BUF_ROWS, DP), jnp.int32),
        scratch_shapes=[pltpu.VMEM((CBLK, D), jnp.float32)],
    )(buf, W1, b1r, W2, b2r)


def _gath_body(g_hbm, c0_hbm, c1_hbm, g0_hbm, g1_hbm, c0_v, c1_v, r0_v, r1_v,
               sem):
    wid = lax.axis_index("s") * 2 + lax.axis_index("c")
    t0 = wid * TPW
    b = t0 // TB
    off = t0 % TB
    pltpu.sync_copy(c0_hbm.at[b, 0, pl.ds(off, TPW)], c0_v)
    pltpu.sync_copy(c1_hbm.at[b, 0, pl.ds(off, TPW)], c1_v)
    a0 = pltpu.async_copy(g_hbm.at[c0_v], r0_v, sem)
    a1 = pltpu.async_copy(g_hbm.at[c1_v], r1_v, sem)
    a0.wait()
    a1.wait()
    pltpu.sync_copy(r0_v, g0_hbm.at[pl.ds(t0, TPW)])
    pltpu.sync_copy(r1_v, g1_hbm.at[pl.ds(t0, TPW)])


def _gather(g, c0, c1):
    mesh = plsc.VectorSubcoreMesh(core_axis_name="c", subcore_axis_name="s")
    out = jax.ShapeDtypeStruct((T, DP), jnp.int32)
    return pl.kernel(
        _gath_body,
        out_type=[out, out],
        mesh=mesh,
        scratch_types=[
            pltpu.VMEM((TPW,), jnp.int32),
            pltpu.VMEM((TPW,), jnp.int32),
            pltpu.VMEM((TPW, DP), jnp.int32),
            pltpu.VMEM((TPW, DP), jnp.int32),
            pltpu.SemaphoreType.DMA,
        ],
    )(g, c0, c1)


def _comb_body(g0_ref, g1_ref, w0_ref, w1_ref, y_ref):
    w0 = jnp.broadcast_to(w0_ref[0], (128, TB)).T[:, 0:1]
    w1 = jnp.broadcast_to(w1_ref[0], (128, TB)).T[:, 0:1]
    y_ref[...] = _unpack(g0_ref[...]) * w0 + _unpack(g1_ref[...]) * w1


def _combine(g0, g1, w0, w1):
    return pl.pallas_call(
        _comb_body,
        grid=(NBLK,),
        in_specs=[
            pl.BlockSpec((TB, DP), lambda i: (i, 0)),
            pl.BlockSpec((TB, DP), lambda i: (i, 0)),
            pl.BlockSpec((1, 1, TB), lambda i: (i, 0, 0)),
            pl.BlockSpec((1, 1, TB), lambda i: (i, 0, 0)),
        ],
        out_specs=pl.BlockSpec((TB, D), lambda i: (i, 0)),
        out_shape=jax.ShapeDtypeStruct((T, D), jnp.float32),
    )(g0, g1, w0, w1)


def kernel(x, Wg, W1, b1, W2, b2):
    wgp = jnp.pad(Wg, ((0, 0), (0, 128 - E)))
    b1r = b1.reshape(E, 1, F)
    b2r = b2.reshape(E, 1, D)
    d0, d1, c0, c1, w0, w1, xp = _router(x, wgp)
    buf = _dispatch(xp, d0, d1)
    g = _ffn(buf, W1, b1r, W2, b2r)
    g0, g1 = _gather(g, c0, c1)
    return _combine(g0, g1, w0, w1)


# speedup vs baseline: 1.1037x; 1.1037x over previous
import functools

import jax
import jax.numpy as jnp
from jax import lax
from jax.experimental import pallas as pl
from jax.experimental.pallas import tpu as pltpu
from jax.experimental.pallas import tpu_sc as plsc

E = 8
K = 2
D = 1024
DP = D // 2
F = 2048
T = 2048
C = 640
EC = E * C
BUF_ROWS = EC + 128
DUMP = EC

TB = 1024
NBLK = T // TB

NW = 32
TPW = T // NW

CBLK = 640


def _b16(x):
    xi = lax.bitcast_convert_type(x, jnp.int32)
    r = xi + jnp.int32(0x7FFF) + ((xi >> 16) & 1)
    return (r >> 16) & jnp.int32(0xFFFF)


def _pack(v):
    return (_b16(v[:, :DP]) << 16) | _b16(v[:, DP:])


def _unpack(p):
    hi = lax.bitcast_convert_type(p & jnp.int32(-65536), jnp.float32)
    lo = lax.bitcast_convert_type(p << 16, jnp.float32)
    return jnp.concatenate([hi, lo], axis=1)


def _router_body(x_ref, wg_ref, d0_ref, d1_ref, c0_ref, c1_ref, w0_ref, w1_ref,
                 xp_ref, carry_ref):
    b = pl.program_id(0)

    @pl.when(b == 0)
    def _init():
        carry_ref[...] = jnp.zeros_like(carry_ref)

    xb = x_ref[...]
    xp_ref[...] = _pack(xb)

    wg = wg_ref[...]
    logits = jnp.dot(xb, wg, preferred_element_type=jnp.float32)
    lt = logits.T
    rows = lax.broadcasted_iota(jnp.int32, (128, TB), 0)
    valid = rows < E

    lm = jnp.where(valid, lt, jnp.float32(-1e30))
    m = jnp.max(lm, axis=0, keepdims=True)
    p = jnp.where(valid, jnp.exp(lt - m), 0.0)
    p = p / jnp.sum(p, axis=0, keepdims=True)

    v1 = jnp.max(jnp.where(valid, p, -1.0), axis=0, keepdims=True)
    i1 = jnp.min(jnp.where(valid & (p == v1), rows, 128), axis=0, keepdims=True)
    m2 = valid & (rows != i1)
    v2 = jnp.max(jnp.where(m2, p, -1.0), axis=0, keepdims=True)
    i2 = jnp.min(jnp.where(m2 & (p == v2), rows, 128), axis=0, keepdims=True)

    ws = v1 + v2 + 1e-9
    w0 = v1 / ws
    w1 = v2 / ws

    oh0 = (rows == i1).astype(jnp.float32)
    oh1 = (rows == i2).astype(jnp.float32)
    oh = oh0 + oh1

    r2 = lax.broadcasted_iota(jnp.int32, (TB, TB), 0)
    c2 = lax.broadcasted_iota(jnp.int32, (TB, TB), 1)
    su = (r2 < c2).astype(jnp.float32)
    base = jnp.dot(oh, su, preferred_element_type=jnp.float32)
    base = base + carry_ref[...]
    carry_ref[...] = carry_ref[...] + jnp.sum(oh, axis=1, keepdims=True)

    pos0 = jnp.sum(oh0 * base, axis=0, keepdims=True).astype(jnp.int32)
    pos1 = jnp.sum(oh1 * base, axis=0, keepdims=True).astype(jnp.int32)
    keep0 = pos0 < C
    keep1 = pos1 < C

    d0 = jnp.where(keep0, i1 * C + pos0, DUMP)
    d1 = jnp.where(keep1, i2 * C + pos1, DUMP)
    c0 = jnp.where(keep0, d0, 0)
    c1 = jnp.where(keep1, d1, 0)
    w0k = jnp.where(keep0, w0, 0.0)
    w1k = jnp.where(keep1, w1, 0.0)

    d0_ref[...] = d0.reshape(1, 1, TB)
    d1_ref[...] = d1.reshape(1, 1, TB)
    c0_ref[...] = c0.reshape(1, 1, TB)
    c1_ref[...] = c1.reshape(1, 1, TB)
    w0_ref[...] = w0k.reshape(1, 1, TB)
    w1_ref[...] = w1k.reshape(1, 1, TB)


def _router(x, wgp):
    out3i = jax.ShapeDtypeStruct((NBLK, 1, TB), jnp.int32)
    out3f = jax.ShapeDtypeStruct((NBLK, 1, TB), jnp.float32)
    spec3 = pl.BlockSpec((1, 1, TB), lambda i: (i, 0, 0))
    return pl.pallas_call(
        _router_body,
        grid=(NBLK,),
        in_specs=[
            pl.BlockSpec((TB, D), lambda i: (i, 0)),
            pl.BlockSpec((D, 128), lambda i: (0, 0)),
        ],
        out_specs=[spec3, spec3, spec3, spec3, spec3, spec3,
                   pl.BlockSpec((TB, DP), lambda i: (i, 0))],
        out_shape=[out3i, out3i, out3i, out3i, out3f, out3f,
                   jax.ShapeDtypeStruct((T, DP), jnp.int32)],
        scratch_shapes=[pltpu.VMEM((128, 1), jnp.float32)],
    )(x, wgp)


def _disp_body(xp_hbm, d0_hbm, d1_hbm, buf_hbm, d0_v, d1_v, rows_v, sem):
    wid = lax.axis_index("s") * 2 + lax.axis_index("c")
    t0 = wid * TPW
    b = t0 // TB
    off = t0 % TB
    pltpu.sync_copy(d0_hbm.at[b, 0, pl.ds(off, TPW)], d0_v)
    pltpu.sync_copy(d1_hbm.at[b, 0, pl.ds(off, TPW)], d1_v)
    pltpu.sync_copy(xp_hbm.at[pl.ds(t0, TPW)], rows_v)
    a0 = pltpu.async_copy(rows_v, buf_hbm.at[d0_v], sem)
    a1 = pltpu.async_copy(rows_v, buf_hbm.at[d1_v], sem)
    a0.wait()
    a1.wait()


def _dispatch(xp, d0, d1):
    mesh = plsc.VectorSubcoreMesh(core_axis_name="c", subcore_axis_name="s")
    return pl.kernel(
        _disp_body,
        out_type=jax.ShapeDtypeStruct((BUF_ROWS, DP), jnp.int32),
        mesh=mesh,
        scratch_types=[
            pltpu.VMEM((TPW,), jnp.int32),
            pltpu.VMEM((TPW,), jnp.int32),
            pltpu.VMEM((TPW, DP), jnp.int32),
            pltpu.SemaphoreType.DMA,
        ],
    )(xp, d0, d1)


def _ffn_body(buf_ref, w1_ref, b1_ref, w2_ref, b2_ref, out_ref):
    xb = _unpack(buf_ref[...])
    h = jnp.dot(xb, w1_ref[0], preferred_element_type=jnp.float32) + b1_ref[0]
    h = jnp.maximum(h, 0.0)
    out = jnp.dot(h, w2_ref[0], preferred_element_type=jnp.float32) + b2_ref[0]
    out_ref[...] = _pack(out)


def _ffn(buf, W1, b1r, W2, b2r):
    return pl.pallas_call(
        _ffn_body,
        grid=(E,),
        in_specs=[
            pl.BlockSpec((CBLK, DP), lambda e: (e, 0)),
            pl.BlockSpec((1, D, F), lambda e: (e, 0, 0)),
            pl.BlockSpec((1, 1, F), lambda e: (e, 0, 0)),
            pl.BlockSpec((1, F, D), lambda e: (e, 0, 0)),
            pl.BlockSpec((1, 1, D), lambda e: (e, 0, 0)),
        ],
        out_specs=pl.BlockSpec((CBLK, DP), lambda e: (e, 0)),
        out_shape=jax.ShapeDtypeStruct((BUF_ROWS, DP), jnp.int32),
    )(buf, W1, b1r, W2, b2r)


def _gath_body(g_hbm, c0_hbm, c1_hbm, g0_hbm, g1_hbm, c0_v, c1_v, r0_v, r1_v,
               sem):
    wid = lax.axis_index("s") * 2 + lax.axis_index("c")
    t0 = wid * TPW
    b = t0 // TB
    off = t0 % TB
    pltpu.sync_copy(c0_hbm.at[b, 0, pl.ds(off, TPW)], c0_v)
    pltpu.sync_copy(c1_hbm.at[b, 0, pl.ds(off, TPW)], c1_v)
    a0 = pltpu.async_copy(g_hbm.at[c0_v], r0_v, sem)
    a1 = pltpu.async_copy(g_hbm.at[c1_v], r1_v, sem)
    a0.wait()
    a1.wait()
    pltpu.sync_copy(r0_v, g0_hbm.at[pl.ds(t0, TPW)])
    pltpu.sync_copy(r1_v, g1_hbm.at[pl.ds(t0, TPW)])


def _gather(g, c0, c1):
    mesh = plsc.VectorSubcoreMesh(core_axis_name="c", subcore_axis_name="s")
    out = jax.ShapeDtypeStruct((T, DP), jnp.int32)
    return pl.kernel(
        _gath_body,
        out_type=[out, out],
        mesh=mesh,
        scratch_types=[
            pltpu.VMEM((TPW,), jnp.int32),
            pltpu.VMEM((TPW,), jnp.int32),
            pltpu.VMEM((TPW, DP), jnp.int32),
            pltpu.VMEM((TPW, DP), jnp.int32),
            pltpu.SemaphoreType.DMA,
        ],
    )(g, c0, c1)


def _comb_body(g0_ref, g1_ref, w0_ref, w1_ref, y_ref):
    w0 = jnp.broadcast_to(w0_ref[0], (128, TB)).T[:, 0:1]
    w1 = jnp.broadcast_to(w1_ref[0], (128, TB)).T[:, 0:1]
    y_ref[...] = _unpack(g0_ref[...]) * w0 + _unpack(g1_ref[...]) * w1


def _combine(g0, g1, w0, w1):
    return pl.pallas_call(
        _comb_body,
        grid=(NBLK,),
        in_specs=[
            pl.BlockSpec((TB, DP), lambda i: (i, 0)),
            pl.BlockSpec((TB, DP), lambda i: (i, 0)),
            pl.BlockSpec((1, 1, TB), lambda i: (i, 0, 0)),
            pl.BlockSpec((1, 1, TB), lambda i: (i, 0, 0)),
        ],
        out_specs=pl.BlockSpec((TB, D), lambda i: (i, 0)),
        out_shape=jax.ShapeDtypeStruct((T, D), jnp.float32),
    )(g0, g1, w0, w1)


def kernel(x, Wg, W1, b1, W2, b2):
    wgp = jnp.pad(Wg, ((0, 0), (0, 128 - E)))
    b1r = b1.reshape(E, 1, F)
    b2r = b2.reshape(E, 1, D)
    d0, d1, c0, c1, w0, w1, xp = _router(x, wgp)
    buf = _dispatch(xp, d0, d1)
    g = _ffn(buf, W1, b1r, W2, b2r)
    g0, g1 = _gather(g, c0, c1)
    return _combine(g0, g1, w0, w1)
